# baseline (speedup 1.0000x reference)
import functools

import jax
import jax.numpy as jnp
from jax import lax
from jax.experimental import pallas as pl
from jax.experimental.pallas import tpu as pltpu
from jax.experimental.pallas import tpu_sc as plsc

N = 10000
E = 320000
D = 128
HD = D // 2
NP = 10240
EP = 327680

NC = 2
NS = 16
NW = NC * NS
K = 80
EPW = E // NW
NCH = EPW // K
KE = 128
KCH = EP // NS // KE
R = NP // NS
ZR = 64

_mesh = plsc.VectorSubcoreMesh(core_axis_name="c", subcore_axis_name="s")


@functools.partial(
    pl.kernel,
    out_type=jax.ShapeDtypeStruct((NC, NP, 16), jnp.float32),
    mesh=_mesh,
    scratch_types=[
        pltpu.VMEM((NCH, K), jnp.int32),
        pltpu.VMEM((K, 16), jnp.float32),
        pltpu.VMEM((R, 16), jnp.float32),
        pltpu.VMEM_SHARED((NP, 16), jnp.float32),
    ],
    compiler_params=pltpu.CompilerParams(use_tc_tiling_on_sc=False),
)
def _deg_sc(dst_hbm, out_hbm, idx_v, ones_v, zero_v, acc_sh):
    c = lax.axis_index("c")
    s = lax.axis_index("s")
    wid = c * NS + s
    pltpu.sync_copy(dst_hbm.at[wid], idx_v)

    def _fill(i, _):
        ones_v[i] = jnp.ones((16,), jnp.float32)
        return 0

    lax.fori_loop(0, K, _fill, 0)

    def _fillz(i, _):
        zero_v[i] = jnp.zeros((16,), jnp.float32)
        return 0

    lax.fori_loop(0, R, _fillz, 0)
    pltpu.sync_copy(zero_v, acc_sh.at[pl.ds(s * R, R)])
    plsc.subcore_barrier()

    def _body(j, _):
        pltpu.sync_copy(ones_v, acc_sh.at[idx_v.at[j]], add=True)
        return 0

    lax.fori_loop(0, NCH, _body, 0)
    plsc.subcore_barrier()
    pltpu.sync_copy(acc_sh.at[pl.ds(s * R, R)], out_hbm.at[c].at[pl.ds(s * R, R)])


@functools.partial(
    pl.kernel,
    out_type=jax.ShapeDtypeStruct((NC, NP, HD), jnp.float32),
    mesh=_mesh,
    scratch_types=[
        pltpu.VMEM((KCH, KE), jnp.int32),
        pltpu.VMEM((KCH, KE), jnp.int32),
        [pltpu.VMEM((KE, HD), jnp.float32)] * 5,
        pltpu.VMEM((ZR, HD), jnp.float32),
        pltpu.VMEM_SHARED((NP, HD), jnp.float32),
        [pltpu.SemaphoreType.DMA] * 5,
        pltpu.SemaphoreType.DMA,
        pltpu.SemaphoreType.DMA,
    ],
    compiler_params=pltpu.CompilerParams(use_tc_tiling_on_sc=False),
)
def _edge_sc(table_hbm, src_hbm, dst_hbm, out_hbm,
             src_v, dst_v, rows, zero_v, acc_sh,
             gsem, isem0, isem1):
    c = lax.axis_index("c")
    s = lax.axis_index("s")
    tb = table_hbm.at[c]
    pltpu.async_copy(src_hbm.at[s], src_v, isem0)
    pltpu.async_copy(dst_hbm.at[s], dst_v, isem1)

    def _fillz(i, _):
        for q in range(HD // 16):
            zero_v[i, pl.ds(q * 16, 16)] = jnp.zeros((16,), jnp.float32)
        return 0

    lax.fori_loop(0, ZR, _fillz, 0)
    for t in range(R // ZR):
        pltpu.sync_copy(zero_v, acc_sh.at[pl.ds(s * R + t * ZR, ZR)])
    pltpu.make_async_copy(src_hbm.at[s], src_v, isem0).wait()
    pltpu.make_async_copy(dst_hbm.at[s], dst_v, isem1).wait()
    plsc.subcore_barrier()

    for p in range(5):
        pltpu.async_copy(tb.at[src_v.at[p]], rows[p], gsem[p])

    def _body(jj, _):
        j0 = 5 * jj
        for p in range(5):
            j = j0 + p
            pltpu.make_async_copy(tb.at[src_v.at[j]], rows[p], gsem[p]).wait()
            pltpu.sync_copy(rows[p], acc_sh.at[dst_v.at[j]], add=True)

            @pl.when(j + 5 < KCH)
            def _():
                pltpu.async_copy(tb.at[src_v.at[j + 5]], rows[p], gsem[p])

        return 0

    lax.fori_loop(0, KCH // 5, _body, 0)
    plsc.subcore_barrier()
    pltpu.sync_copy(acc_sh.at[pl.ds(s * R, R)], out_hbm.at[c].at[pl.ds(s * R, R)])


_BR = 1024


def _split_cols(m):
    return jnp.stack([m[:, :HD], m[:, HD:]])


def _tc_scale_body(x_ref, degp_ref, xnt_ref, normb_ref):
    deg = degp_ref[0, :, 0:1] + degp_ref[1, :, 0:1] + 1.0
    norm = lax.rsqrt(deg)
    nb = jnp.broadcast_to(norm, (_BR, D))
    normb_ref[...] = nb
    xnt_ref[...] = _split_cols(x_ref[...] * nb)


def _tc_mid_body(sp_ref, x_ref, normb_ref, b_ref, w_ref, x1_ref, xnt_ref):
    nb = normb_ref[...]
    sfull = jnp.concatenate([sp_ref[0], sp_ref[1]], axis=1)
    u = nb * sfull + (nb * nb) * x_ref[...]
    x1 = jnp.maximum(
        jnp.dot(u, w_ref[...], preferred_element_type=jnp.float32)
        + b_ref[...], 0.0)
    x1_ref[...] = x1
    xnt_ref[...] = _split_cols(x1 * nb)


def _tc_post_body(sp_ref, x_ref, normb_ref, b_ref, w_ref, x2_ref):
    nb = normb_ref[...]
    sfull = jnp.concatenate([sp_ref[0], sp_ref[1]], axis=1)
    u = nb * sfull + (nb * nb) * x_ref[...]
    x2_ref[...] = jnp.maximum(
        jnp.dot(u, w_ref[...], preferred_element_type=jnp.float32)
        + b_ref[...], 0.0)


_row_spec = pl.BlockSpec((_BR, D), lambda i: (i, 0))
_t_spec = pl.BlockSpec((NC, _BR, HD), lambda i: (0, i, 0))
_dgp_spec = pl.BlockSpec((NC, _BR, 16), lambda i: (0, i, 0))
_w_spec = pl.BlockSpec((D, D), lambda i: (0, 0))
_b_spec = pl.BlockSpec((1, D), lambda i: (0, 0))
_grid = (NP // _BR,)

_row_shape = jax.ShapeDtypeStruct((NP, D), jnp.float32)
_t_shape = jax.ShapeDtypeStruct((NC, NP, HD), jnp.float32)

_tc_scale = pl.pallas_call(
    _tc_scale_body,
    grid=_grid,
    in_specs=[_row_spec, _dgp_spec],
    out_specs=[_t_spec, _row_spec],
    out_shape=[_t_shape, _row_shape],
)

_tc_mid = pl.pallas_call(
    _tc_mid_body,
    grid=_grid,
    in_specs=[_t_spec, _row_spec, _row_spec, _b_spec, _w_spec],
    out_specs=[_row_spec, _t_spec],
    out_shape=[_row_shape, _t_shape],
)

_tc_post = pl.pallas_call(
    _tc_post_body,
    grid=_grid,
    in_specs=[_t_spec, _row_spec, _row_spec, _b_spec, _w_spec],
    out_specs=_row_spec,
    out_shape=_row_shape,
)


def kernel(data, edge_index, W1, b1, W2, b2):
    srcd = edge_index[0].reshape(NW, NCH, K)
    dstd = edge_index[1].reshape(NW, NCH, K)
    epad = EP - E
    srce = jnp.pad(edge_index[0], (0, epad),
                   constant_values=NP - 1).reshape(NS, KCH, KE)
    dste = jnp.pad(edge_index[1], (0, epad),
                   constant_values=NP - 1).reshape(NS, KCH, KE)
    data_p = jnp.pad(data, ((0, NP - N), (0, 0)))
    degp = _deg_sc(dstd)
    xnt1, normb = _tc_scale(data_p, degp)
    s1 = _edge_sc(xnt1, srce, dste)
    x1, xnt2 = _tc_mid(s1, data_p, normb, b1.reshape(1, D), W1)
    s2 = _edge_sc(xnt2, srce, dste)
    x2 = _tc_post(s2, x1, normb, b2.reshape(1, D), W2)
    return (x2[:N], x1[:N], x2[:N])


# speedup vs baseline: 2.1138x; 2.1138x over previous
import functools

import jax
import jax.numpy as jnp
from jax import lax
from jax.experimental import pallas as pl
from jax.experimental.pallas import tpu as pltpu
from jax.experimental.pallas import tpu_sc as plsc

N = 10000
E = 320000
D = 128
HD = D // 2
NP = 10240
EP = 327680

NC = 2
NS = 16
NW = NC * NS
K = 80
EPW = E // NW
NCH = EPW // K
KE = 80
KCH = E // NS // KE
R = NP // NS
ZR = 64

_mesh = plsc.VectorSubcoreMesh(core_axis_name="c", subcore_axis_name="s")


@functools.partial(
    pl.kernel,
    out_type=jax.ShapeDtypeStruct((NC, NP, 16), jnp.float32),
    mesh=_mesh,
    scratch_types=[
        pltpu.VMEM((NCH, K), jnp.int32),
        pltpu.VMEM((K, 16), jnp.float32),
        pltpu.VMEM((R, 16), jnp.float32),
        pltpu.VMEM_SHARED((NP, 16), jnp.float32),
    ],
    compiler_params=pltpu.CompilerParams(use_tc_tiling_on_sc=False),
)
def _deg_sc(dst_hbm, out_hbm, idx_v, ones_v, zero_v, acc_sh):
    c = lax.axis_index("c")
    s = lax.axis_index("s")
    wid = c * NS + s
    pltpu.sync_copy(dst_hbm.at[wid], idx_v)

    def _fill(i, _):
        ones_v[i] = jnp.ones((16,), jnp.float32)
        return 0

    lax.fori_loop(0, K, _fill, 0)

    def _fillz(i, _):
        zero_v[i] = jnp.zeros((16,), jnp.float32)
        return 0

    lax.fori_loop(0, R, _fillz, 0)
    pltpu.sync_copy(zero_v, acc_sh.at[pl.ds(s * R, R)])
    plsc.subcore_barrier()

    def _body(j, _):
        pltpu.sync_copy(ones_v, acc_sh.at[idx_v.at[j]], add=True)
        return 0

    lax.fori_loop(0, NCH, _body, 0)
    plsc.subcore_barrier()
    pltpu.sync_copy(acc_sh.at[pl.ds(s * R, R)], out_hbm.at[c].at[pl.ds(s * R, R)])


@functools.partial(
    pl.kernel,
    out_type=jax.ShapeDtypeStruct((NC, NP, HD), jnp.float32),
    mesh=_mesh,
    scratch_types=[
        pltpu.VMEM((KCH, KE), jnp.int32),
        pltpu.VMEM((KCH, KE), jnp.int32),
        [pltpu.VMEM((KE, HD), jnp.float32)] * 5,
        pltpu.VMEM((ZR, HD), jnp.float32),
        pltpu.VMEM_SHARED((NP, HD), jnp.float32),
        [pltpu.SemaphoreType.DMA] * 5,
        pltpu.SemaphoreType.DMA,
        pltpu.SemaphoreType.DMA,
    ],
    compiler_params=pltpu.CompilerParams(use_tc_tiling_on_sc=False),
)
def _edge_sc(table_hbm, src_hbm, dst_hbm, out_hbm,
             src_v, dst_v, rows, zero_v, acc_sh,
             gsem, isem0, isem1):
    c = lax.axis_index("c")
    s = lax.axis_index("s")
    tb = table_hbm.at[c]
    pltpu.async_copy(src_hbm.at[s], src_v, isem0)
    pltpu.async_copy(dst_hbm.at[s], dst_v, isem1)

    def _fillz(i, _):
        for q in range(HD // 16):
            zero_v[i, pl.ds(q * 16, 16)] = jnp.zeros((16,), jnp.float32)
        return 0

    lax.fori_loop(0, ZR, _fillz, 0)
    for t in range(R // ZR):
        pltpu.sync_copy(zero_v, acc_sh.at[pl.ds(s * R + t * ZR, ZR)])
    pltpu.make_async_copy(src_hbm.at[s], src_v, isem0).wait()
    pltpu.make_async_copy(dst_hbm.at[s], dst_v, isem1).wait()
    plsc.subcore_barrier()

    for p in range(5):
        pltpu.async_copy(tb.at[src_v.at[p]], rows[p], gsem[p])

    def _body(jj, _):
        j0 = 5 * jj
        for p in range(5):
            j = j0 + p
            pltpu.make_async_copy(tb.at[src_v.at[j]], rows[p], gsem[p]).wait()
            pltpu.sync_copy(rows[p], acc_sh.at[dst_v.at[j]], add=True)

            @pl.when(j + 5 < KCH)
            def _():
                pltpu.async_copy(tb.at[src_v.at[j + 5]], rows[p], gsem[p])

        return 0

    lax.fori_loop(0, KCH // 5, _body, 0)
    plsc.subcore_barrier()
    pltpu.sync_copy(acc_sh.at[pl.ds(s * R, R)], out_hbm.at[c].at[pl.ds(s * R, R)])


_BR = 1024


def _split_cols(m):
    return jnp.stack([m[:, :HD], m[:, HD:]])


def _tc_scale_body(x_ref, degp_ref, xnt_ref, normb_ref):
    deg = degp_ref[0, :, 0:1] + degp_ref[1, :, 0:1] + 1.0
    norm = lax.rsqrt(deg)
    nb = jnp.broadcast_to(norm, (_BR, D))
    normb_ref[...] = nb
    xnt_ref[...] = _split_cols(x_ref[...] * nb)


def _tc_mid_body(sp_ref, x_ref, normb_ref, b_ref, w_ref, x1_ref, xnt_ref):
    nb = normb_ref[...]
    sfull = jnp.concatenate([sp_ref[0], sp_ref[1]], axis=1)
    u = nb * sfull + (nb * nb) * x_ref[...]
    x1 = jnp.maximum(
        jnp.dot(u, w_ref[...], preferred_element_type=jnp.float32)
        + b_ref[...], 0.0)
    x1_ref[...] = x1
    xnt_ref[...] = _split_cols(x1 * nb)


def _tc_post_body(sp_ref, x_ref, normb_ref, b_ref, w_ref, x2_ref):
    nb = normb_ref[...]
    sfull = jnp.concatenate([sp_ref[0], sp_ref[1]], axis=1)
    u = nb * sfull + (nb * nb) * x_ref[...]
    x2_ref[...] = jnp.maximum(
        jnp.dot(u, w_ref[...], preferred_element_type=jnp.float32)
        + b_ref[...], 0.0)


_row_spec = pl.BlockSpec((_BR, D), lambda i: (i, 0))
_t_spec = pl.BlockSpec((NC, _BR, HD), lambda i: (0, i, 0))
_dgp_spec = pl.BlockSpec((NC, _BR, 16), lambda i: (0, i, 0))
_w_spec = pl.BlockSpec((D, D), lambda i: (0, 0))
_b_spec = pl.BlockSpec((1, D), lambda i: (0, 0))
_grid = (NP // _BR,)

_row_shape = jax.ShapeDtypeStruct((NP, D), jnp.float32)
_t_shape = jax.ShapeDtypeStruct((NC, NP, HD), jnp.float32)

_tc_scale = pl.pallas_call(
    _tc_scale_body,
    grid=_grid,
    in_specs=[_row_spec, _dgp_spec],
    out_specs=[_t_spec, _row_spec],
    out_shape=[_t_shape, _row_shape],
)

_tc_mid = pl.pallas_call(
    _tc_mid_body,
    grid=_grid,
    in_specs=[_t_spec, _row_spec, _row_spec, _b_spec, _w_spec],
    out_specs=[_row_spec, _t_spec],
    out_shape=[_row_shape, _t_shape],
)

_tc_post = pl.pallas_call(
    _tc_post_body,
    grid=_grid,
    in_specs=[_t_spec, _row_spec, _row_spec, _b_spec, _w_spec],
    out_specs=_row_spec,
    out_shape=_row_shape,
)


def kernel(data, edge_index, W1, b1, W2, b2):
    srcd = edge_index[0].reshape(NW, NCH, K)
    dstd = edge_index[1].reshape(NW, NCH, K)
    srce = edge_index[0].reshape(NS, KCH, KE)
    dste = edge_index[1].reshape(NS, KCH, KE)
    data_p = jnp.pad(data, ((0, NP - N), (0, 0)))
    degp = _deg_sc(dstd)
    xnt1, normb = _tc_scale(data_p, degp)
    s1 = _edge_sc(xnt1, srce, dste)
    x1, xnt2 = _tc_mid(s1, data_p, normb, b1.reshape(1, D), W1)
    s2 = _edge_sc(xnt2, srce, dste)
    x2 = _tc_post(s2, x1, normb, b2.reshape(1, D), W2)
    return (x2[:N], x1[:N], x2[:N])
